# baseline (speedup 1.0000x reference)
import jax
import jax.numpy as jnp
from jax import lax
from jax.experimental import pallas as pl
from jax.experimental.pallas import tpu as pltpu
from jax.experimental.pallas import tpu_sc as plsc

L = 16
NC = 2
NS = 16
NW = NC * NS
B = 128
N = 32768
NV = N // L
K = 30
RPW = B // NW
NB = 256
CAP1 = 640
CAP2 = 32

import numpy as np

I32 = jnp.int32
IMIN = np.int32(-(2 ** 31))
IMAX = np.int32(2 ** 31 - 1)
MANT = np.int32(0x7FFFFFFF)


def _body(pred_hbm, target_hbm, out_hbm,
          row_v, ck_v, ci_v, hist_v, tot_v, c2k_v, c2i_v, out_v):
    iota = lax.iota(I32, L)
    ones = jnp.ones((L,), I32)
    zero16i = jnp.zeros((L,), I32)

    wid = lax.axis_index("s") * NC + lax.axis_index("c")

    def f2key(x):
        b = plsc.bitcast(x, I32)
        return b ^ ((b >> 31) & MANT)

    def key2f(k):
        return plsc.bitcast(k ^ ((k >> 31) & MANT), jnp.float32)

    def extract(vec, j, fill):
        return jnp.max(jnp.where(iota == j, vec, fill))

    def suffix_search(svec, kthr):
        vsuffix = lax.rev(plsc.cumsum(lax.rev(svec, (0,))), (0,))
        ngt = jnp.sum((vsuffix >= kthr).astype(I32))
        return ngt - 1, vsuffix

    def bucket_totals():
        def tv(v, svec):
            base = v * 256 + iota * 16
            t = zero16i
            for l in range(L):
                t = t + plsc.load_gather(hist_v, [base + l])
            tot_v[pl.ds(v * 16, 16)] = t
            return jnp.where(iota == v, jnp.sum(t), svec)
        return lax.fori_loop(0, 16, tv, zero16i)

    def zero_hist():
        def z(i, c):
            for u in range(8):
                hist_v[pl.ds((i * 8 + u) * 16, 16)] = zero16i
            return c
        lax.fori_loop(0, NB // 8, z, 0)

    def topk_row(src_hbm, r, pad_idx):
        pltpu.sync_copy(src_hbm.at[r], row_v)
        zero_hist()

        def p1(i, c):
            for u in range(4):
                x = row_v[pl.ds((i * 4 + u) * 16, 16)]
                k = f2key(x)
                addr = (((k >> 24) + 128) << 4) + iota
                plsc.addupdate_scatter(hist_v, [addr], ones)
            return c
        lax.fori_loop(0, NV // 4, p1, 0)

        svec = bucket_totals()
        vstar, vsfx = suffix_search(svec, K)
        above_v = extract(vsfx, vstar, IMIN) - extract(svec, vstar, IMIN)
        t = tot_v[pl.ds(vstar * 16, 16)]
        sfx_in = lax.rev(plsc.cumsum(lax.rev(t, (0,))), (0,)) + above_v
        npos = jnp.sum((sfx_in >= K).astype(I32))
        b1_sel = vstar * 16 + npos - 1
        c_above = extract(sfx_in, npos - 1, IMIN) - extract(t, npos - 1, IMIN)
        t1 = (b1_sel - 128) << 24

        def p2(i, cnt):
            for u in range(4):
                ii = i * 4 + u
                k = f2key(row_v[pl.ds(ii * 16, 16)])
                m = (k >= t1) & (cnt < CAP1)
                pos = (cnt << 4) + iota
                plsc.store_scatter(ck_v, [pos], k, mask=m)
                plsc.store_scatter(ci_v, [pos], (ii << 4) + iota, mask=m)
                cnt = cnt + m.astype(I32)
            return cnt
        cnt1 = lax.fori_loop(0, NV // 4, p2, zero16i)
        maxc1 = jnp.max(cnt1)

        zero_hist()
        b1s = b1_sel - 128

        def p2b(c, z):
            k = ck_v[pl.ds(c * 16, 16)]
            m = (c < cnt1) & ((k >> 24) == b1s)
            addr = (((k >> 16) & 255) << 4) + iota
            plsc.addupdate_scatter(hist_v, [addr], ones, mask=m)
            return z
        lax.fori_loop(0, maxc1, p2b, 0)

        k2 = K - c_above
        svec2 = bucket_totals()
        vstar2, vsfx2 = suffix_search(svec2, k2)
        above_v2 = extract(vsfx2, vstar2, IMIN) - extract(svec2, vstar2, IMIN)
        t2 = tot_v[pl.ds(vstar2 * 16, 16)]
        sfx_in2 = lax.rev(plsc.cumsum(lax.rev(t2, (0,))), (0,)) + above_v2
        npos2 = jnp.sum((sfx_in2 >= k2).astype(I32))
        s_sel = vstar2 * 16 + npos2 - 1
        thr = t1 + (s_sel << 16)

        def p2c(c, cnt):
            k = ck_v[pl.ds(c * 16, 16)]
            idv = ci_v[pl.ds(c * 16, 16)]
            m = (c < cnt1) & (k >= thr) & (cnt < CAP2)
            pos = (cnt << 4) + iota
            plsc.store_scatter(c2k_v, [pos], k, mask=m)
            plsc.store_scatter(c2i_v, [pos], idv, mask=m)
            return cnt + m.astype(I32)
        cnt2 = lax.fori_loop(0, maxc1, p2c, zero16i)
        maxc2 = jnp.max(cnt2)

        def sel_iter(j, carry):
            pk, pi, sk0, sk1, si0, si1 = carry

            def srow(c, bc):
                bk, bi = bc
                k = c2k_v[pl.ds(c * 16, 16)]
                idv = c2i_v[pl.ds(c * 16, 16)]
                valid = (c < cnt2) & ((k < pk) | ((k == pk) & (idv > pi)))
                k = jnp.where(valid, k, IMIN)
                idv = jnp.where(valid, idv, IMAX)
                better = (k > bk) | ((k == bk) & (idv < bi))
                return (jnp.where(better, k, bk), jnp.where(better, idv, bi))

            bk, bi = lax.fori_loop(0, maxc2, srow,
                                   (jnp.full((L,), IMIN), jnp.full((L,), IMAX)))
            mk = jnp.max(bk)
            mi = jnp.min(jnp.where(bk == mk, bi, IMAX))
            m0 = iota == j
            m1 = iota == (j - 16)
            return (mk, mi,
                    jnp.where(m0, mk, sk0), jnp.where(m1, mk, sk1),
                    jnp.where(m0, mi, si0), jnp.where(m1, mi, si1))

        init = (IMAX, jnp.int32(-1), zero16i, zero16i,
                jnp.full((L,), jnp.int32(pad_idx)), jnp.full((L,), jnp.int32(pad_idx)))
        _, _, sk0, sk1, si0, si1 = lax.fori_loop(0, K, sel_iter, init)
        return sk0, sk1, si0, si1

    def per_row(i, c):
        r = wid * RPW + i
        tk0, tk1, ti0, ti1 = topk_row(target_hbm, r, -2)
        pk0, pk1, pi0, pi1 = topk_row(pred_hbm, r, -1)

        d0 = jnp.abs(key2f(pk0) - key2f(tk0))
        d1 = jnp.abs(key2f(pk1) - key2f(tk1))
        rowsum = jnp.sum(d0) + jnp.sum(d1)

        def inter(j, acc):
            m0, m1 = acc
            e0 = extract(ti0, j, IMIN)
            e1 = extract(ti1, j - 16, IMIN)
            tj = jnp.maximum(e0, e1)
            return (m0 | (pi0 == tj), m1 | (pi1 == tj))

        m0, m1 = lax.fori_loop(0, K, inter,
                               (jnp.zeros((L,), jnp.bool_), jnp.zeros((L,), jnp.bool_)))
        nmatch = jnp.sum(m0.astype(jnp.float32)) + jnp.sum(m1.astype(jnp.float32))

        outrow = jnp.where(iota == 0, nmatch,
                           jnp.where(iota == 1, rowsum, jnp.float32(0.0)))
        out_v[pl.ds(i * 16, 16)] = outrow
        return c

    lax.fori_loop(0, RPW, per_row, 0)
    pltpu.sync_copy(out_v, out_hbm.at[wid])


_mesh = plsc.VectorSubcoreMesh(core_axis_name="c", subcore_axis_name="s",
                               num_cores=NC, num_subcores=NS)

_sc_call = pl.kernel(
    _body,
    out_type=jax.ShapeDtypeStruct((NW, RPW * L), jnp.float32),
    mesh=_mesh,
    compiler_params=pltpu.CompilerParams(needs_layout_passes=False),
    scratch_types=[
        pltpu.VMEM((N,), jnp.float32),
        pltpu.VMEM((CAP1 * L,), I32),
        pltpu.VMEM((CAP1 * L,), I32),
        pltpu.VMEM((NB * L,), I32),
        pltpu.VMEM((NB,), I32),
        pltpu.VMEM((CAP2 * L,), I32),
        pltpu.VMEM((CAP2 * L,), I32),
        pltpu.VMEM((RPW * L,), jnp.float32),
    ],
)


def kernel(pred, target):
    out = _sc_call(pred, target)
    a = out.reshape(B, L)
    counts = a[:, 0]
    rowsums = a[:, 1]
    s1 = counts.sum() / jnp.float32(B * K)
    s2 = rowsums.sum() / jnp.float32(K)
    return (s1, s2)


# speedup vs baseline: 2.5396x; 2.5396x over previous
import jax
import jax.numpy as jnp
import numpy as np
from jax import lax
from jax.experimental import pallas as pl
from jax.experimental.pallas import tpu as pltpu
from jax.experimental.pallas import tpu_sc as plsc

L = 16
NC = 2
NS = 16
NW = NC * NS
B = 128
N = 32768
NV = N // L
K = 30
RPW = B // NW
NB = 256
NREG = 4
CAP1 = 640
CAP2 = 32

I32 = jnp.int32
IMIN = np.int32(-(2 ** 31))
IMAX = np.int32(2 ** 31 - 1)
MANT = np.int32(0x7FFFFFFF)


def _body(pred_hbm, target_hbm, out_hbm,
          row_v, ck_v, ci_v, hist_v, tot_v, c2k_v, c2i_v, ti_v, out_v):
    iota = lax.iota(I32, L)
    ones = jnp.ones((L,), I32)
    zero16i = jnp.zeros((L,), I32)

    wid = lax.axis_index("s") * NC + lax.axis_index("c")

    def f2key(x):
        b = plsc.bitcast(x, I32)
        return b ^ ((b >> 31) & MANT)

    def key2f(k):
        return plsc.bitcast(k ^ ((k >> 31) & MANT), jnp.float32)

    def extract(vec, j, fill):
        return jnp.max(jnp.where(iota == j, vec, fill))

    def suffix_search(svec, kthr):
        vsuffix = lax.rev(plsc.cumsum(lax.rev(svec, (0,))), (0,))
        ngt = jnp.sum((vsuffix >= kthr).astype(I32))
        return ngt - 1, vsuffix

    def bucket_totals(nreg):
        def tv(v, svec):
            base = v * 256 + iota * 16
            acc = [zero16i for _ in range(4)]
            for u in range(nreg):
                for l in range(L):
                    acc[l % 4] = acc[l % 4] + plsc.load_gather(
                        hist_v, [base + ((u << 12) + l)])
            t = (acc[0] + acc[1]) + (acc[2] + acc[3])
            tot_v[pl.ds(v * 16, 16)] = t
            return jnp.where(iota == v, jnp.sum(t), svec)
        return lax.fori_loop(0, 16, tv, zero16i)

    def zero_hist(nreg):
        @plsc.parallel_loop(0, nreg * NB, 1, unroll=8)
        def z(i):
            hist_v[pl.ds(i * 16, 16)] = zero16i

    def topk_row(src_hbm, r):
        pltpu.sync_copy(src_hbm.at[r], row_v)
        zero_hist(NREG)

        @plsc.parallel_loop(0, NV, 1, unroll=4)
        def p1(i):
            x = row_v[pl.ds(i * 16, 16)]
            k = f2key(x)
            addr = ((i & 3) << 12) + (((k >> 24) + 128) << 4) + iota
            plsc.addupdate_scatter(hist_v, [addr], ones)

        svec = bucket_totals(NREG)
        vstar, vsfx = suffix_search(svec, K)
        above_v = extract(vsfx, vstar, IMIN) - extract(svec, vstar, IMIN)
        t = tot_v[pl.ds(vstar * 16, 16)]
        sfx_in = lax.rev(plsc.cumsum(lax.rev(t, (0,))), (0,)) + above_v
        npos = jnp.sum((sfx_in >= K).astype(I32))
        b1_sel = vstar * 16 + npos - 1
        c_above = extract(sfx_in, npos - 1, IMIN) - extract(t, npos - 1, IMIN)
        t1 = (b1_sel - 128) << 24

        @plsc.parallel_loop(0, NV, 1, unroll=4, carry=zero16i)
        def p2(i, cnt):
            k = f2key(row_v[pl.ds(i * 16, 16)])
            m = (k >= t1) & (cnt < CAP1)
            pos = (cnt << 4) + iota
            plsc.store_scatter(ck_v, [pos], k, mask=m)
            plsc.store_scatter(ci_v, [pos], (i << 4) + iota, mask=m)
            return cnt + m.astype(I32)
        cnt1 = p2
        maxc1 = jnp.max(cnt1)

        zero_hist(1)
        b1s = b1_sel - 128

        def p2b(c, z):
            k = ck_v[pl.ds(c * 16, 16)]
            m = (c < cnt1) & ((k >> 24) == b1s)
            addr = (((k >> 16) & 255) << 4) + iota
            plsc.addupdate_scatter(hist_v, [addr], ones, mask=m)
            return z
        lax.fori_loop(0, maxc1, p2b, 0)

        k2 = K - c_above
        svec2 = bucket_totals(1)
        vstar2, vsfx2 = suffix_search(svec2, k2)
        above_v2 = extract(vsfx2, vstar2, IMIN) - extract(svec2, vstar2, IMIN)
        t2 = tot_v[pl.ds(vstar2 * 16, 16)]
        sfx_in2 = lax.rev(plsc.cumsum(lax.rev(t2, (0,))), (0,)) + above_v2
        npos2 = jnp.sum((sfx_in2 >= k2).astype(I32))
        s_sel = vstar2 * 16 + npos2 - 1
        thr = t1 + (s_sel << 16)

        @plsc.parallel_loop(0, maxc1, 1, unroll=2, carry=zero16i)
        def p2c(c, cnt):
            k = ck_v[pl.ds(c * 16, 16)]
            idv = ci_v[pl.ds(c * 16, 16)]
            m = (c < cnt1) & (k >= thr) & (cnt < CAP2)
            pos = (cnt << 4) + iota
            plsc.store_scatter(c2k_v, [pos], k, mask=m)
            plsc.store_scatter(c2i_v, [pos], idv, mask=m)
            return cnt + m.astype(I32)
        cnt2 = p2c
        maxc2 = jnp.max(cnt2)

        def sel_iter(j, carry):
            pk, pi, sk0, sk1, si0, si1 = carry

            @plsc.parallel_loop(0, maxc2, 1, unroll=2,
                                carry=(jnp.full((L,), IMIN), jnp.full((L,), IMAX)))
            def srow(c, bc):
                bk, bi = bc
                k = c2k_v[pl.ds(c * 16, 16)]
                idv = c2i_v[pl.ds(c * 16, 16)]
                valid = (c < cnt2) & ((k < pk) | ((k == pk) & (idv > pi)))
                k = jnp.where(valid, k, IMIN)
                idv = jnp.where(valid, idv, IMAX)
                better = (k > bk) | ((k == bk) & (idv < bi))
                return (jnp.where(better, k, bk), jnp.where(better, idv, bi))

            bk, bi = srow
            mk = jnp.max(bk)
            mi = jnp.min(jnp.where(bk == mk, bi, IMAX))
            m0 = iota == j
            m1 = iota == (j - 16)
            return (mk, mi,
                    jnp.where(m0, mk, sk0), jnp.where(m1, mk, sk1),
                    jnp.where(m0, mi, si0), jnp.where(m1, mi, si1))

        init = (jnp.int32(IMAX), jnp.int32(-1), zero16i, zero16i,
                zero16i - 1, zero16i - 1)
        _, _, sk0, sk1, si0, si1 = lax.fori_loop(0, K, sel_iter, init)
        return sk0, sk1, si0, si1

    def per_row(i, c):
        r = wid * RPW + i
        tk0, tk1, ti0, ti1 = topk_row(target_hbm, r)
        ti_v[pl.ds(0, 16)] = ti0
        ti_v[pl.ds(16, 16)] = ti1
        pk0, pk1, pi0, pi1 = topk_row(pred_hbm, r)

        d0 = jnp.abs(key2f(pk0) - key2f(tk0))
        d1 = jnp.abs(key2f(pk1) - key2f(tk1))
        rowsum = jnp.sum(d0) + jnp.sum(d1)

        def inter(j, acc):
            m0, m1 = acc
            tj = plsc.load_gather(ti_v, [zero16i + j])
            return (m0 | (pi0 == tj), m1 | (pi1 == tj))

        m0, m1 = lax.fori_loop(0, K, inter,
                               (jnp.zeros((L,), jnp.bool_),
                                jnp.zeros((L,), jnp.bool_)))
        nmatch = jnp.sum(m0.astype(jnp.float32)) + jnp.sum(m1.astype(jnp.float32))

        outrow = jnp.where(iota == 0, nmatch,
                           jnp.where(iota == 1, rowsum, jnp.float32(0.0)))
        out_v[pl.ds(i * 16, 16)] = outrow
        return c

    lax.fori_loop(0, RPW, per_row, 0)
    pltpu.sync_copy(out_v, out_hbm.at[wid])


_mesh = plsc.VectorSubcoreMesh(core_axis_name="c", subcore_axis_name="s",
                               num_cores=NC, num_subcores=NS)

_sc_call = pl.kernel(
    _body,
    out_type=jax.ShapeDtypeStruct((NW, RPW * L), jnp.float32),
    mesh=_mesh,
    compiler_params=pltpu.CompilerParams(needs_layout_passes=False),
    scratch_types=[
        pltpu.VMEM((N,), jnp.float32),
        pltpu.VMEM((CAP1 * L,), I32),
        pltpu.VMEM((CAP1 * L,), I32),
        pltpu.VMEM((NREG * NB * L,), I32),
        pltpu.VMEM((NB,), I32),
        pltpu.VMEM((CAP2 * L,), I32),
        pltpu.VMEM((CAP2 * L,), I32),
        pltpu.VMEM((2 * L,), I32),
        pltpu.VMEM((RPW * L,), jnp.float32),
    ],
)


def kernel(pred, target):
    out = _sc_call(pred, target)
    a = out.reshape(B, L)
    counts = a[:, 0]
    rowsums = a[:, 1]
    s1 = counts.sum() / jnp.float32(B * K)
    s2 = rowsums.sum() / jnp.float32(K)
    return (s1, s2)
